# Initial kernel scaffold; baseline (speedup 1.0000x reference)
import jax
import jax.numpy as jnp
from jax.experimental import pallas as pl


def kernel(skills, embedding_table):
    raise NotImplementedError("write your pallas kernel here")


# speedup vs baseline: 6.0650x; 6.0650x over previous
import functools

import jax
import jax.numpy as jnp
from jax import lax
from jax.experimental import pallas as pl
from jax.experimental.pallas import tpu as pltpu
from jax.experimental.pallas import tpu_sc as plsc

EMBED_DIM = 32
NUM_CORES = 2
NUM_SUBCORES = 16
NUM_WORKERS = NUM_CORES * NUM_SUBCORES
GRP = 128
K = 10
CHUNK = K * GRP

_MESH = plsc.VectorSubcoreMesh(
    core_axis_name="c", subcore_axis_name="s",
    num_cores=NUM_CORES, num_subcores=NUM_SUBCORES,
)


def _make_gather(total_rows: int, nchunks: int):
  @functools.partial(
      pl.kernel,
      mesh=_MESH,
      compiler_params=pltpu.CompilerParams(use_tc_tiling_on_sc=False),
      out_type=jax.ShapeDtypeStruct(
          (NUM_WORKERS, nchunks, K, GRP, EMBED_DIM), jnp.float32),
      scratch_types=[
          pltpu.VMEM((K, GRP), jnp.int32),
          pltpu.VMEM((K, GRP, EMBED_DIM), jnp.float32),
          pltpu.SemaphoreType.DMA,
      ],
  )
  def gather_kernel(idx_hbm, table_hbm, out_hbm, idx_v, rows_v, gsem):
    wid = lax.axis_index("s") * NUM_CORES + lax.axis_index("c")

    @pl.loop(0, nchunks)
    def _chunk(g):
      pltpu.sync_copy(idx_hbm.at[wid, g], idx_v)
      for j in range(K):
        pltpu.async_copy(table_hbm.at[idx_v.at[j]], rows_v.at[j], gsem)
      for j in range(K):
        pltpu.make_async_copy(table_hbm.at[idx_v.at[j]], rows_v.at[j],
                              gsem).wait()
      pltpu.sync_copy(rows_v, out_hbm.at[wid, g])

  return gather_kernel


def kernel(skills, embedding_table):
  batch, hist = skills.shape
  total = batch * hist
  assert total % (NUM_WORKERS * CHUNK) == 0
  nchunks = total // (NUM_WORKERS * CHUNK)
  idx = skills.reshape(NUM_WORKERS, nchunks, K, GRP)
  out = _make_gather(total, nchunks)(idx, embedding_table)
  return out.reshape(batch, hist, EMBED_DIM)


# speedup vs baseline: 6.2919x; 1.0374x over previous
import functools

import jax
import jax.numpy as jnp
from jax import lax
from jax.experimental import pallas as pl
from jax.experimental.pallas import tpu as pltpu
from jax.experimental.pallas import tpu_sc as plsc

EMBED_DIM = 32
NUM_CORES = 2
NUM_SUBCORES = 16
NUM_WORKERS = NUM_CORES * NUM_SUBCORES
GRP = 128
K = 10
CHUNK = K * GRP

_MESH = plsc.VectorSubcoreMesh(
    core_axis_name="c", subcore_axis_name="s",
    num_cores=NUM_CORES, num_subcores=NUM_SUBCORES,
)


def _make_gather(nchunks: int):
  @functools.partial(
      pl.kernel,
      mesh=_MESH,
      compiler_params=pltpu.CompilerParams(use_tc_tiling_on_sc=False),
      out_type=jax.ShapeDtypeStruct(
          (NUM_WORKERS, nchunks, K, GRP, EMBED_DIM), jnp.float32),
      scratch_types=[
          pltpu.VMEM((nchunks * K, GRP), jnp.int32),
          pltpu.VMEM((2, K, GRP, EMBED_DIM), jnp.float32),
          pltpu.SemaphoreType.DMA,
          pltpu.SemaphoreType.DMA,
      ],
  )
  def gather_kernel(idx_hbm, table_hbm, out_hbm, idx_v, rows_v, gsem, osem):
    wid = lax.axis_index("s") * NUM_CORES + lax.axis_index("c")

    def fire(g, slot):
      for j in range(K):
        pltpu.async_copy(table_hbm.at[idx_v.at[g * K + j]],
                         rows_v.at[slot, j], gsem)

    def drain_gathers(slot):
      pltpu.make_async_copy(out_hbm.at[wid, 0], rows_v.at[slot], gsem).wait()

    def drain_write():
      pltpu.make_async_copy(rows_v.at[0], out_hbm.at[wid, 0], osem).wait()

    pltpu.sync_copy(idx_hbm.at[wid], idx_v)
    fire(0, 0)

    @pl.loop(0, nchunks)
    def _chunk(g):
      s = g % 2
      has_next = g + 1 < nchunks

      @pl.when(jnp.logical_and(g >= 1, has_next))
      def _():
        drain_write()

      @pl.when(has_next)
      def _():
        fire(g + 1, 1 - s)

      drain_gathers(s)
      pltpu.async_copy(rows_v.at[s], out_hbm.at[wid, g], osem)

    drain_write()
    drain_write()

  return gather_kernel


def kernel(skills, embedding_table):
  batch, hist = skills.shape
  total = batch * hist
  assert total % (NUM_WORKERS * CHUNK) == 0
  nchunks = total // (NUM_WORKERS * CHUNK)
  idx = skills.reshape(NUM_WORKERS, nchunks * K, GRP)
  out = _make_gather(nchunks)(idx, embedding_table)
  return out.reshape(batch, hist, EMBED_DIM)
